# baseline (speedup 1.0000x reference)
import jax
import jax.numpy as jnp
from jax import lax
from jax.experimental import pallas as pl
from jax.experimental.pallas import tpu as pltpu
from jax.experimental.pallas import tpu_sc as plsc

_NUM_LIGHTING = 16
_NUM_SAMPLE = 262144
_NUM_CHANNEL = 3
_ROW = _NUM_SAMPLE * _NUM_CHANNEL
_NUM_CORES = 2
_NUM_SUBCORES = 16
_NW = _NUM_CORES * _NUM_SUBCORES
_CHUNK = _ROW // _NW


def _copy_row_body(table_hbm, idx_hbm, out_hbm, idx_v, buf, sem_in, sem_out):
    wid = lax.axis_index("s") * _NUM_CORES + lax.axis_index("c")
    base = wid * _CHUNK
    pltpu.sync_copy(idx_hbm, idx_v)
    row = idx_v[...][0]
    pltpu.async_copy(
        table_hbm.at[row, pl.ds(base, _CHUNK)], buf, sem_in
    ).wait()
    pltpu.async_copy(buf, out_hbm.at[pl.ds(base, _CHUNK)], sem_out).wait()


def kernel(l_samples, l_dir, lighting_idx):
    del l_dir
    table = l_samples.reshape(_NUM_LIGHTING, _ROW)
    idx16 = jnp.full((16,), lighting_idx, dtype=jnp.int32)

    mesh = plsc.VectorSubcoreMesh(core_axis_name="c", subcore_axis_name="s")
    flat = pl.kernel(
        _copy_row_body,
        out_type=jax.ShapeDtypeStruct((_ROW,), jnp.float32),
        mesh=mesh,
        scratch_types=[
            pltpu.VMEM((16,), jnp.int32),
            pltpu.VMEM((_CHUNK,), jnp.float32),
            pltpu.SemaphoreType.DMA,
            pltpu.SemaphoreType.DMA,
        ],
    )(table, idx16)
    return flat.reshape(1, _NUM_SAMPLE, _NUM_CHANNEL)


# speedup vs baseline: 30.7996x; 30.7996x over previous
import jax
import jax.numpy as jnp
from jax import lax
from jax.experimental import pallas as pl
from jax.experimental.pallas import tpu as pltpu
from jax.experimental.pallas import tpu_sc as plsc

_NUM_LIGHTING = 16
_NUM_SAMPLE = 262144
_NUM_CHANNEL = 3
_LANES = 128
_SUBLANES = 8
_TROW = _NUM_LIGHTING // _SUBLANES
_TCOL = _NUM_SAMPLE // _LANES
_NUM_CORES = 2
_NUM_SUBCORES = 16
_NW = _NUM_CORES * _NUM_SUBCORES
_JCHUNK = _TCOL // _NW


def _copy_row_body(tab, idx_hbm, out, idx_v, buf, sem_in, sem_out):
    wid = lax.axis_index("s") * _NUM_CORES + lax.axis_index("c")
    j0 = wid * _JCHUNK
    pltpu.sync_copy(idx_hbm, idx_v)
    row = idx_v[...][0]
    trow = lax.shift_right_logical(row, 3)
    sub = lax.bitwise_and(row, 7)
    for c in range(_NUM_CHANNEL):
        pltpu.async_copy(
            tab.at[c, trow, pl.ds(j0, _JCHUNK), sub, :], buf.at[c], sem_in
        ).wait()
        pltpu.async_copy(
            buf.at[c], out.at[c, pl.ds(j0, _JCHUNK), :], sem_out
        ).wait()


def kernel(l_samples, l_dir, lighting_idx):
    del l_dir
    tab = (
        l_samples.transpose(2, 0, 1)
        .reshape(_NUM_CHANNEL, _TROW, _SUBLANES, _TCOL, _LANES)
        .transpose(0, 1, 3, 2, 4)
    )
    idx16 = jnp.full((16,), lighting_idx, dtype=jnp.int32)

    mesh = plsc.VectorSubcoreMesh(core_axis_name="c", subcore_axis_name="s")
    out = pl.kernel(
        _copy_row_body,
        out_type=jax.ShapeDtypeStruct((_NUM_CHANNEL, _TCOL, _LANES), jnp.float32),
        mesh=mesh,
        scratch_types=[
            pltpu.VMEM((16,), jnp.int32),
            pltpu.VMEM((_NUM_CHANNEL, _JCHUNK, _LANES), jnp.float32),
            pltpu.SemaphoreType.DMA,
            pltpu.SemaphoreType.DMA,
        ],
    )(tab, idx16)
    return out.reshape(_NUM_CHANNEL, _NUM_SAMPLE).T[None]


# speedup vs baseline: 42.8492x; 1.3912x over previous
import jax
import jax.numpy as jnp
from jax import lax
from jax.experimental import pallas as pl
from jax.experimental.pallas import tpu as pltpu
from jax.experimental.pallas import tpu_sc as plsc

_NUM_LIGHTING = 16
_NUM_SAMPLE = 262144
_NUM_CHANNEL = 3
_LANES = 128
_SUBLANES = 8
_TROW = _NUM_LIGHTING // _SUBLANES
_TCOL = _NUM_SAMPLE // _LANES
_NUM_CORES = 2
_NUM_SUBCORES = 16
_NW = _NUM_CORES * _NUM_SUBCORES
_JCHUNK = _TCOL // _NW


def _copy_row_body(tab, idx_hbm, out, idx_v, buf, sem_in, sem_out):
    wid = lax.axis_index("s") * _NUM_CORES + lax.axis_index("c")
    j0 = wid * _JCHUNK
    pltpu.sync_copy(idx_hbm, idx_v)
    row = idx_v[...][0]
    trow = lax.shift_right_logical(row, 3)
    sub = lax.bitwise_and(row, 7)
    for c in range(_NUM_CHANNEL):
        pltpu.async_copy(
            tab.at[c, trow, pl.ds(j0, _JCHUNK), sub, :], buf.at[c], sem_in
        ).wait()
        pltpu.async_copy(
            buf.at[c], out.at[c, 0, pl.ds(j0, _JCHUNK), :], sem_out
        ).wait()


def kernel(l_samples, l_dir, lighting_idx):
    del l_dir
    tab = (
        l_samples.transpose(2, 0, 1)
        .reshape(_NUM_CHANNEL, _TROW, _SUBLANES, _TCOL, _LANES)
        .transpose(0, 1, 3, 2, 4)
    )
    idx16 = jnp.full((16,), lighting_idx, dtype=jnp.int32)

    mesh = plsc.VectorSubcoreMesh(core_axis_name="c", subcore_axis_name="s")
    out = pl.kernel(
        _copy_row_body,
        out_type=jax.ShapeDtypeStruct((_NUM_CHANNEL, 1, _TCOL, _LANES), jnp.float32),
        mesh=mesh,
        scratch_types=[
            pltpu.VMEM((16,), jnp.int32),
            pltpu.VMEM((_NUM_CHANNEL, _JCHUNK, _LANES), jnp.float32),
            pltpu.SemaphoreType.DMA,
            pltpu.SemaphoreType.DMA,
        ],
    )(tab, idx16)
    return out.reshape(_NUM_CHANNEL, 1, _NUM_SAMPLE).transpose(1, 2, 0)


# speedup vs baseline: 46.5363x; 1.0860x over previous
import jax
import jax.numpy as jnp
from jax import lax
from jax.experimental import pallas as pl
from jax.experimental.pallas import tpu as pltpu
from jax.experimental.pallas import tpu_sc as plsc

_NUM_LIGHTING = 16
_NUM_SAMPLE = 262144
_NUM_CHANNEL = 3
_LANES = 128
_SUBLANES = 8
_TROW = _NUM_LIGHTING // _SUBLANES
_TCOL = _NUM_SAMPLE // _LANES
_NUM_CORES = 2
_NUM_SUBCORES = 16
_NW = _NUM_CORES * _NUM_SUBCORES
_JCHUNK = _TCOL // _NW


def _copy_row_body(tab, idx_hbm, out, idx_v, buf, sem_in, sem_out):
    wid = lax.axis_index("s") * _NUM_CORES + lax.axis_index("c")
    j0 = wid * _JCHUNK
    pltpu.sync_copy(idx_hbm, idx_v)
    row = idx_v[...][0]
    trow = lax.shift_right_logical(row, 3)
    sub = lax.bitwise_and(row, 7)
    pltpu.async_copy(
        tab.at[:, trow, pl.ds(j0, _JCHUNK), sub, :], buf, sem_in
    ).wait()
    pltpu.async_copy(
        buf, out.at[:, 0, pl.ds(j0, _JCHUNK), :], sem_out
    ).wait()


def kernel(l_samples, l_dir, lighting_idx):
    del l_dir
    tab = (
        l_samples.transpose(2, 0, 1)
        .reshape(_NUM_CHANNEL, _TROW, _SUBLANES, _TCOL, _LANES)
        .transpose(0, 1, 3, 2, 4)
    )
    idx16 = jnp.full((16,), lighting_idx, dtype=jnp.int32)

    mesh = plsc.VectorSubcoreMesh(core_axis_name="c", subcore_axis_name="s")
    out = pl.kernel(
        _copy_row_body,
        out_type=jax.ShapeDtypeStruct((_NUM_CHANNEL, 1, _TCOL, _LANES), jnp.float32),
        mesh=mesh,
        scratch_types=[
            pltpu.VMEM((16,), jnp.int32),
            pltpu.VMEM((_NUM_CHANNEL, _JCHUNK, _LANES), jnp.float32),
            pltpu.SemaphoreType.DMA,
            pltpu.SemaphoreType.DMA,
        ],
    )(tab, idx16)
    return out.reshape(_NUM_CHANNEL, 1, _NUM_SAMPLE).transpose(1, 2, 0)
